# baseline (speedup 1.0000x reference)
import functools

import jax
import jax.numpy as jnp
from jax import lax
from jax.experimental import pallas as pl
from jax.experimental.pallas import tpu as pltpu
from jax.experimental.pallas import tpu_sc as plsc

N = 10000
E = 320000
D = 128
H = 128
O = 64
G = 16

NC = 2
NS = 16
NW = NC * NS
CH = 128
NCH = 79
NCH0 = 97
NCH1 = 61
EPAD = NW * NCH * CH
NPAD = N + 112
RPS = NPAD // NS
HN = NCH // 2

_MESH = plsc.VectorSubcoreMesh(core_axis_name="c", subcore_axis_name="s")


def _deg_body(dst_hbm, ones_hbm, zeros_hbm, out_hbm, dst_v, ones_v, acc_sh):
    c = lax.axis_index("c")
    s = lax.axis_index("s")
    w = c * NS + s
    pltpu.sync_copy(zeros_hbm.at[pl.ds(s * RPS, RPS)], acc_sh.at[pl.ds(s * RPS, RPS)])
    pltpu.sync_copy(dst_hbm.at[w], dst_v)
    pltpu.sync_copy(ones_hbm, ones_v)
    plsc.subcore_barrier()

    @pl.loop(0, NCH)
    def _(j):
        pltpu.sync_copy(ones_v, acc_sh.at[dst_v.at[j]], add=True)

    plsc.subcore_barrier()
    pltpu.sync_copy(acc_sh.at[pl.ds(s * RPS, RPS)], out_hbm.at[c, pl.ds(s * RPS, RPS)])


def _sc_degree(dst3, ones128, zeros128):
    kern = pl.kernel(
        _deg_body,
        out_type=jax.ShapeDtypeStruct((NC, NPAD, D), jnp.float32),
        mesh=_MESH,
        scratch_types=[
            pltpu.VMEM((NCH, CH), jnp.int32),
            pltpu.VMEM((CH, D), jnp.float32),
            pltpu.VMEM_SHARED((NPAD, D), jnp.float32),
        ],
    )
    return kern(dst3, ones128, zeros128)


def _agg_body(src_hbm, dst_hbm, h_hbm, zeros_hbm, out_hbm,
              src_v, dst_v, rows_v, acc_sh):
    c = lax.axis_index("c")
    s = lax.axis_index("s")
    w = c * NS + s
    pltpu.sync_copy(zeros_hbm.at[pl.ds(s * RPS, RPS)], acc_sh.at[pl.ds(s * RPS, RPS)])
    pltpu.sync_copy(src_hbm.at[w], src_v)
    pltpu.sync_copy(dst_hbm.at[w], dst_v)
    plsc.subcore_barrier()

    @pl.when(c == 0)
    def _():
        @pl.loop(0, NCH0)
        def _(j):
            pltpu.sync_copy(h_hbm.at[src_v.at[j]], rows_v)
            pltpu.sync_copy(rows_v, acc_sh.at[dst_v.at[j]], add=True)

    @pl.when(c == 1)
    def _():
        @pl.loop(0, NCH1)
        def _(j):
            pltpu.sync_copy(h_hbm.at[src_v.at[j]], rows_v)
            pltpu.sync_copy(rows_v, acc_sh.at[dst_v.at[j]], add=True)

    plsc.subcore_barrier()
    pltpu.sync_copy(acc_sh.at[pl.ds(s * RPS, RPS)], out_hbm.at[c, pl.ds(s * RPS, RPS)])


def _sc_aggregate(src3, dst3, h, zeros128):
    kern = pl.kernel(
        _agg_body,
        out_type=jax.ShapeDtypeStruct((NC, NPAD, D), jnp.float32),
        mesh=_MESH,
        scratch_types=[
            pltpu.VMEM((NCH0, CH), jnp.int32),
            pltpu.VMEM((NCH0, CH), jnp.int32),
            pltpu.VMEM((CH, D), jnp.float32),
            pltpu.VMEM_SHARED((NPAD, D), jnp.float32),
        ],
    )
    return kern(src3, dst3, h, zeros128)


_BLK = 2528


def _mm_body(x_ref, w_ref, o_ref):
    o_ref[...] = jnp.dot(x_ref[...], w_ref[...], preferred_element_type=jnp.float32)


def _tc_matmul(x, w):
    return pl.pallas_call(
        _mm_body,
        grid=(NPAD // _BLK,),
        in_specs=[
            pl.BlockSpec((_BLK, D), lambda i: (i, 0)),
            pl.BlockSpec((D, H), lambda i: (0, 0)),
        ],
        out_specs=pl.BlockSpec((_BLK, H), lambda i: (i, 0)),
        out_shape=jax.ShapeDtypeStruct((NPAD, H), jnp.float32),
    )(x, w)


def _prep_body(deg2_ref, h1_ref, dinv_ref, h1p_ref):
    deg = deg2_ref[0][:, :16] + deg2_ref[1][:, :16] + 1.0
    dinv = lax.rsqrt(deg)
    dinv_ref[...] = dinv
    h1p_ref[...] = dinv[:, :1] * h1_ref[...]


def _tc_prep(deg2, h1):
    return pl.pallas_call(
        _prep_body,
        out_shape=(
            jax.ShapeDtypeStruct((NPAD, 16), jnp.float32),
            jax.ShapeDtypeStruct((NPAD, H), jnp.float32),
        ),
    )(deg2, h1)


def _mid_body(part_ref, h1_ref, dinv_ref, w2_ref, b1_ref, h2_ref, h2p_ref):
    dinv = dinv_ref[...][:, :1]
    agg = part_ref[0] + part_ref[1]
    z1 = jnp.maximum(dinv * agg + dinv * dinv * h1_ref[...] + b1_ref[...], 0.0)
    h2 = jnp.dot(z1, w2_ref[...], preferred_element_type=jnp.float32)
    h2_ref[...] = h2
    h2p_ref[...] = dinv * h2


def _tc_mid(part1, h1, dinv16, w2, b1):
    return pl.pallas_call(
        _mid_body,
        grid=(NPAD // _BLK,),
        in_specs=[
            pl.BlockSpec((NC, _BLK, H), lambda i: (0, i, 0)),
            pl.BlockSpec((_BLK, H), lambda i: (i, 0)),
            pl.BlockSpec((_BLK, 16), lambda i: (i, 0)),
            pl.BlockSpec((H, H), lambda i: (0, 0)),
            pl.BlockSpec((1, H), lambda i: (0, 0)),
        ],
        out_specs=(
            pl.BlockSpec((_BLK, H), lambda i: (i, 0)),
            pl.BlockSpec((_BLK, H), lambda i: (i, 0)),
        ),
        out_shape=(
            jax.ShapeDtypeStruct((NPAD, H), jnp.float32),
            jax.ShapeDtypeStruct((NPAD, H), jnp.float32),
        ),
    )(part1, h1, dinv16, w2, b1)


def _final_body(part_ref, h2_ref, dinv_ref, b2_ref, batch_ref, wfc_ref, bfc_ref,
                out_ref, acc_ref, cnt_ref):
    i = pl.program_id(0)

    @pl.when(i == 0)
    def _():
        acc_ref[...] = jnp.zeros_like(acc_ref)
        cnt_ref[...] = jnp.zeros_like(cnt_ref)

    dinv = dinv_ref[...][:, :1]
    z2 = dinv * (part_ref[0] + part_ref[1]) + dinv * dinv * h2_ref[...] + b2_ref[...]
    onehot = (batch_ref[...] == lax.broadcasted_iota(jnp.int32, (1, G), 1)
              ).astype(jnp.float32)
    acc_ref[...] += lax.dot_general(onehot, z2, (((0,), (0,)), ((), ())),
                                    preferred_element_type=jnp.float32)
    cnt_ref[...] += lax.dot_general(onehot, jnp.ones_like(z2),
                                    (((0,), (0,)), ((), ())),
                                    preferred_element_type=jnp.float32)

    @pl.when(i == pl.num_programs(0) - 1)
    def _():
        pooled = acc_ref[...] / jnp.maximum(cnt_ref[...], 1.0)
        out_ref[...] = jnp.dot(pooled, wfc_ref[...],
                               preferred_element_type=jnp.float32) + bfc_ref[...]


def _tc_final(part2, h2, dinv16, b2, batch2d, wfc, bfc):
    return pl.pallas_call(
        _final_body,
        grid=(NPAD // _BLK,),
        in_specs=[
            pl.BlockSpec((NC, _BLK, H), lambda i: (0, i, 0)),
            pl.BlockSpec((_BLK, H), lambda i: (i, 0)),
            pl.BlockSpec((_BLK, 16), lambda i: (i, 0)),
            pl.BlockSpec((1, H), lambda i: (0, 0)),
            pl.BlockSpec((_BLK, 1), lambda i: (i, 0)),
            pl.BlockSpec((H, O), lambda i: (0, 0)),
            pl.BlockSpec((1, O), lambda i: (0, 0)),
        ],
        out_specs=pl.BlockSpec((G, O), lambda i: (0, 0)),
        out_shape=jax.ShapeDtypeStruct((G, O), jnp.float32),
        scratch_shapes=[
            pltpu.VMEM((G, H), jnp.float32),
            pltpu.VMEM((G, H), jnp.float32),
        ],
    )(part2, h2, dinv16, b2, batch2d, wfc, bfc)


def kernel(x, edge_index, batch, W1, b1, W2, b2, Wfc, bfc):
    src = edge_index[0]
    dst = edge_index[1]
    pad = EPAD - E
    src_f = jnp.concatenate([src, jnp.zeros((pad,), jnp.int32)])
    dst_f = jnp.concatenate([dst, jnp.full((pad,), N, jnp.int32)])
    dst3 = dst_f.reshape(NW, NCH, CH)

    def agg_layout(flat):
        c0 = flat[: NS * NCH0 * CH].reshape(NS, NCH0, CH)
        c1 = flat[NS * NCH0 * CH :].reshape(NS, NCH1, CH)
        c1 = jnp.pad(c1, ((0, 0), (0, NCH0 - NCH1), (0, 0)))
        return jnp.concatenate([c0, c1], axis=0)

    src3a = agg_layout(src_f)
    dst3a = agg_layout(dst_f)

    x_pad = jnp.pad(x, ((0, NPAD - N), (0, 0)))
    batch2d = jnp.pad(batch, (0, NPAD - N), constant_values=G).reshape(NPAD, 1)
    ones128 = jnp.ones((CH, D), jnp.float32)
    zeros128 = jnp.zeros((NPAD, D), jnp.float32)
    b1r = b1.reshape(1, H)
    b2r = b2.reshape(1, H)
    bfcr = bfc.reshape(1, O)

    deg2 = _sc_degree(dst3, ones128, zeros128)
    h1 = _tc_matmul(x_pad, W1)
    dinv16, h1p = _tc_prep(deg2, h1)
    part1 = _sc_aggregate(src3a, dst3a, h1p, zeros128)
    h2, h2p = _tc_mid(part1, h1, dinv16, W2, b1r)
    part2 = _sc_aggregate(src3a, dst3a, h2p, zeros128)
    return _tc_final(part2, h2, dinv16, b2r, batch2d, Wfc, bfcr)


# speedup vs baseline: 1.1811x; 1.1811x over previous
import functools

import jax
import jax.numpy as jnp
from jax import lax
from jax.experimental import pallas as pl
from jax.experimental.pallas import tpu as pltpu
from jax.experimental.pallas import tpu_sc as plsc

N = 10000
E = 320000
D = 128
H = 128
O = 64
G = 16

NC = 2
NS = 16
NW = NC * NS
CH = 128
NCH = 79
NCH0 = 103
NCH1 = 55
EPAD = NW * NCH * CH
NPAD = N + 112
RPS = NPAD // NS
HN = NCH // 2

_MESH = plsc.VectorSubcoreMesh(core_axis_name="c", subcore_axis_name="s")


def _deg_body(dst_hbm, ones_hbm, zeros_hbm, out_hbm, dst_v, ones_v, acc_sh):
    c = lax.axis_index("c")
    s = lax.axis_index("s")
    w = c * NS + s
    pltpu.sync_copy(zeros_hbm.at[pl.ds(s * RPS, RPS)], acc_sh.at[pl.ds(s * RPS, RPS)])
    pltpu.sync_copy(dst_hbm.at[w], dst_v)
    pltpu.sync_copy(ones_hbm, ones_v)
    plsc.subcore_barrier()

    @pl.loop(0, NCH)
    def _(j):
        pltpu.sync_copy(ones_v, acc_sh.at[dst_v.at[j]], add=True)

    plsc.subcore_barrier()
    pltpu.sync_copy(acc_sh.at[pl.ds(s * RPS, RPS)], out_hbm.at[c, pl.ds(s * RPS, RPS)])


def _sc_degree(dst3, ones128, zeros128):
    kern = pl.kernel(
        _deg_body,
        out_type=jax.ShapeDtypeStruct((NC, NPAD, D), jnp.float32),
        mesh=_MESH,
        scratch_types=[
            pltpu.VMEM((NCH, CH), jnp.int32),
            pltpu.VMEM((CH, D), jnp.float32),
            pltpu.VMEM_SHARED((NPAD, D), jnp.float32),
        ],
    )
    return kern(dst3, ones128, zeros128)


def _agg_body(src_hbm, dst_hbm, h_hbm, zeros_hbm, out_hbm,
              src_v, dst_v, rows_v, acc_sh):
    c = lax.axis_index("c")
    s = lax.axis_index("s")
    w = c * NS + s
    pltpu.sync_copy(zeros_hbm.at[pl.ds(s * RPS, RPS)], acc_sh.at[pl.ds(s * RPS, RPS)])
    pltpu.sync_copy(src_hbm.at[w], src_v)
    pltpu.sync_copy(dst_hbm.at[w], dst_v)
    plsc.subcore_barrier()

    @pl.when(c == 0)
    def _():
        @pl.loop(0, NCH0)
        def _(j):
            pltpu.sync_copy(h_hbm.at[src_v.at[j]], rows_v)
            pltpu.sync_copy(rows_v, acc_sh.at[dst_v.at[j]], add=True)

    @pl.when(c == 1)
    def _():
        @pl.loop(0, NCH1)
        def _(j):
            pltpu.sync_copy(h_hbm.at[src_v.at[j]], rows_v)
            pltpu.sync_copy(rows_v, acc_sh.at[dst_v.at[j]], add=True)

    plsc.subcore_barrier()
    pltpu.sync_copy(acc_sh.at[pl.ds(s * RPS, RPS)], out_hbm.at[c, pl.ds(s * RPS, RPS)])


def _sc_aggregate(src3, dst3, h, zeros128):
    kern = pl.kernel(
        _agg_body,
        out_type=jax.ShapeDtypeStruct((NC, NPAD, D), jnp.float32),
        mesh=_MESH,
        scratch_types=[
            pltpu.VMEM((NCH0, CH), jnp.int32),
            pltpu.VMEM((NCH0, CH), jnp.int32),
            pltpu.VMEM((CH, D), jnp.float32),
            pltpu.VMEM_SHARED((NPAD, D), jnp.float32),
        ],
    )
    return kern(src3, dst3, h, zeros128)


_BLK = 2528


def _mm_body(x_ref, w_ref, o_ref):
    o_ref[...] = jnp.dot(x_ref[...], w_ref[...], preferred_element_type=jnp.float32)


def _tc_matmul(x, w):
    return pl.pallas_call(
        _mm_body,
        grid=(NPAD // _BLK,),
        in_specs=[
            pl.BlockSpec((_BLK, D), lambda i: (i, 0)),
            pl.BlockSpec((D, H), lambda i: (0, 0)),
        ],
        out_specs=pl.BlockSpec((_BLK, H), lambda i: (i, 0)),
        out_shape=jax.ShapeDtypeStruct((NPAD, H), jnp.float32),
    )(x, w)


def _prep_body(deg2_ref, h1_ref, dinv_ref, h1p_ref):
    deg = deg2_ref[0][:, :16] + deg2_ref[1][:, :16] + 1.0
    dinv = lax.rsqrt(deg)
    dinv_ref[...] = dinv
    h1p_ref[...] = dinv[:, :1] * h1_ref[...]


def _tc_prep(deg2, h1):
    return pl.pallas_call(
        _prep_body,
        out_shape=(
            jax.ShapeDtypeStruct((NPAD, 16), jnp.float32),
            jax.ShapeDtypeStruct((NPAD, H), jnp.float32),
        ),
    )(deg2, h1)


def _mid_body(part_ref, h1_ref, dinv_ref, w2_ref, b1_ref, h2_ref, h2p_ref):
    dinv = dinv_ref[...][:, :1]
    agg = part_ref[0] + part_ref[1]
    z1 = jnp.maximum(dinv * agg + dinv * dinv * h1_ref[...] + b1_ref[...], 0.0)
    h2 = jnp.dot(z1, w2_ref[...], preferred_element_type=jnp.float32)
    h2_ref[...] = h2
    h2p_ref[...] = dinv * h2


def _tc_mid(part1, h1, dinv16, w2, b1):
    return pl.pallas_call(
        _mid_body,
        grid=(NPAD // _BLK,),
        in_specs=[
            pl.BlockSpec((NC, _BLK, H), lambda i: (0, i, 0)),
            pl.BlockSpec((_BLK, H), lambda i: (i, 0)),
            pl.BlockSpec((_BLK, 16), lambda i: (i, 0)),
            pl.BlockSpec((H, H), lambda i: (0, 0)),
            pl.BlockSpec((1, H), lambda i: (0, 0)),
        ],
        out_specs=(
            pl.BlockSpec((_BLK, H), lambda i: (i, 0)),
            pl.BlockSpec((_BLK, H), lambda i: (i, 0)),
        ),
        out_shape=(
            jax.ShapeDtypeStruct((NPAD, H), jnp.float32),
            jax.ShapeDtypeStruct((NPAD, H), jnp.float32),
        ),
    )(part1, h1, dinv16, w2, b1)


def _final_body(part_ref, h2_ref, dinv_ref, b2_ref, batch_ref, wfc_ref, bfc_ref,
                out_ref, acc_ref, cnt_ref):
    i = pl.program_id(0)

    @pl.when(i == 0)
    def _():
        acc_ref[...] = jnp.zeros_like(acc_ref)
        cnt_ref[...] = jnp.zeros_like(cnt_ref)

    dinv = dinv_ref[...][:, :1]
    z2 = dinv * (part_ref[0] + part_ref[1]) + dinv * dinv * h2_ref[...] + b2_ref[...]
    onehot = (batch_ref[...] == lax.broadcasted_iota(jnp.int32, (1, G), 1)
              ).astype(jnp.float32)
    acc_ref[...] += lax.dot_general(onehot, z2, (((0,), (0,)), ((), ())),
                                    preferred_element_type=jnp.float32)
    cnt_ref[...] += lax.dot_general(onehot, jnp.ones_like(z2),
                                    (((0,), (0,)), ((), ())),
                                    preferred_element_type=jnp.float32)

    @pl.when(i == pl.num_programs(0) - 1)
    def _():
        pooled = acc_ref[...] / jnp.maximum(cnt_ref[...], 1.0)
        out_ref[...] = jnp.dot(pooled, wfc_ref[...],
                               preferred_element_type=jnp.float32) + bfc_ref[...]


def _tc_final(part2, h2, dinv16, b2, batch2d, wfc, bfc):
    return pl.pallas_call(
        _final_body,
        grid=(NPAD // _BLK,),
        in_specs=[
            pl.BlockSpec((NC, _BLK, H), lambda i: (0, i, 0)),
            pl.BlockSpec((_BLK, H), lambda i: (i, 0)),
            pl.BlockSpec((_BLK, 16), lambda i: (i, 0)),
            pl.BlockSpec((1, H), lambda i: (0, 0)),
            pl.BlockSpec((_BLK, 1), lambda i: (i, 0)),
            pl.BlockSpec((H, O), lambda i: (0, 0)),
            pl.BlockSpec((1, O), lambda i: (0, 0)),
        ],
        out_specs=pl.BlockSpec((G, O), lambda i: (0, 0)),
        out_shape=jax.ShapeDtypeStruct((G, O), jnp.float32),
        scratch_shapes=[
            pltpu.VMEM((G, H), jnp.float32),
            pltpu.VMEM((G, H), jnp.float32),
        ],
    )(part2, h2, dinv16, b2, batch2d, wfc, bfc)


def kernel(x, edge_index, batch, W1, b1, W2, b2, Wfc, bfc):
    src = edge_index[0]
    dst = edge_index[1]
    pad = EPAD - E
    src_f = jnp.concatenate([src, jnp.zeros((pad,), jnp.int32)])
    dst_f = jnp.concatenate(
        [dst, N + (jnp.arange(pad, dtype=jnp.int32) % (NPAD - N))])
    dst3 = dst_f.reshape(NW, NCH, CH)

    def agg_layout(flat):
        c0 = flat[: NS * NCH0 * CH].reshape(NS, NCH0, CH)
        c1 = flat[NS * NCH0 * CH :].reshape(NS, NCH1, CH)
        c1 = jnp.pad(c1, ((0, 0), (0, NCH0 - NCH1), (0, 0)))
        return jnp.concatenate([c0, c1], axis=0)

    src3a = agg_layout(src_f)
    dst3a = agg_layout(dst_f)

    x_pad = jnp.pad(x, ((0, NPAD - N), (0, 0)))
    batch2d = jnp.pad(batch, (0, NPAD - N), constant_values=G).reshape(NPAD, 1)
    ones128 = jnp.ones((CH, D), jnp.float32)
    zeros128 = jnp.zeros((NPAD, D), jnp.float32)
    b1r = b1.reshape(1, H)
    b2r = b2.reshape(1, H)
    bfcr = bfc.reshape(1, O)

    deg2 = _sc_degree(dst3, ones128, zeros128)
    h1 = _tc_matmul(x_pad, W1)
    dinv16, h1p = _tc_prep(deg2, h1)
    part1 = _sc_aggregate(src3a, dst3a, h1p, zeros128)
    h2, h2p = _tc_mid(part1, h1, dinv16, W2, b1r)
    part2 = _sc_aggregate(src3a, dst3a, h2p, zeros128)
    return _tc_final(part2, h2, dinv16, b2r, batch2d, Wfc, bfcr)


# speedup vs baseline: 1.1923x; 1.0095x over previous
import functools

import jax
import jax.numpy as jnp
from jax import lax
from jax.experimental import pallas as pl
from jax.experimental.pallas import tpu as pltpu
from jax.experimental.pallas import tpu_sc as plsc

N = 10000
E = 320000
D = 128
H = 128
O = 64
G = 16

NC = 2
NS = 16
NW = NC * NS
CH = 128
NCH = 79
NCH0 = 107
NCH1 = 51
EPAD = NW * NCH * CH
NPAD = N + 112
RPS = NPAD // NS
HN = NCH // 2

_MESH = plsc.VectorSubcoreMesh(core_axis_name="c", subcore_axis_name="s")


def _deg_body(dst_hbm, ones_hbm, zeros_hbm, out_hbm, dst_v, ones_v, acc_sh):
    c = lax.axis_index("c")
    s = lax.axis_index("s")
    w = c * NS + s
    pltpu.sync_copy(zeros_hbm.at[pl.ds(s * RPS, RPS)], acc_sh.at[pl.ds(s * RPS, RPS)])
    pltpu.sync_copy(dst_hbm.at[w], dst_v)
    pltpu.sync_copy(ones_hbm, ones_v)
    plsc.subcore_barrier()

    @pl.loop(0, NCH)
    def _(j):
        pltpu.sync_copy(ones_v, acc_sh.at[dst_v.at[j]], add=True)

    plsc.subcore_barrier()
    pltpu.sync_copy(acc_sh.at[pl.ds(s * RPS, RPS)], out_hbm.at[c, pl.ds(s * RPS, RPS)])


def _sc_degree(dst3, ones128, zeros128):
    kern = pl.kernel(
        _deg_body,
        out_type=jax.ShapeDtypeStruct((NC, NPAD, D), jnp.float32),
        mesh=_MESH,
        scratch_types=[
            pltpu.VMEM((NCH, CH), jnp.int32),
            pltpu.VMEM((CH, D), jnp.float32),
            pltpu.VMEM_SHARED((NPAD, D), jnp.float32),
        ],
    )
    return kern(dst3, ones128, zeros128)


def _agg_body(src_hbm, dst_hbm, h_hbm, zeros_hbm, out_hbm,
              src_v, dst_v, rows_v, acc_sh):
    c = lax.axis_index("c")
    s = lax.axis_index("s")
    w = c * NS + s
    pltpu.sync_copy(zeros_hbm.at[pl.ds(s * RPS, RPS)], acc_sh.at[pl.ds(s * RPS, RPS)])
    pltpu.sync_copy(src_hbm.at[w], src_v)
    pltpu.sync_copy(dst_hbm.at[w], dst_v)
    plsc.subcore_barrier()

    @pl.when(c == 0)
    def _():
        @pl.loop(0, NCH0)
        def _(j):
            pltpu.sync_copy(h_hbm.at[src_v.at[j]], rows_v)
            pltpu.sync_copy(rows_v, acc_sh.at[dst_v.at[j]], add=True)

    @pl.when(c == 1)
    def _():
        @pl.loop(0, NCH1)
        def _(j):
            pltpu.sync_copy(h_hbm.at[src_v.at[j]], rows_v)
            pltpu.sync_copy(rows_v, acc_sh.at[dst_v.at[j]], add=True)

    plsc.subcore_barrier()
    pltpu.sync_copy(acc_sh.at[pl.ds(s * RPS, RPS)], out_hbm.at[c, pl.ds(s * RPS, RPS)])


def _sc_aggregate(src3, dst3, h, zeros128):
    kern = pl.kernel(
        _agg_body,
        out_type=jax.ShapeDtypeStruct((NC, NPAD, D), jnp.float32),
        mesh=_MESH,
        scratch_types=[
            pltpu.VMEM((NCH0, CH), jnp.int32),
            pltpu.VMEM((NCH0, CH), jnp.int32),
            pltpu.VMEM((CH, D), jnp.float32),
            pltpu.VMEM_SHARED((NPAD, D), jnp.float32),
        ],
    )
    return kern(src3, dst3, h, zeros128)


_BLK = 2528


def _mm_body(x_ref, w_ref, o_ref):
    o_ref[...] = jnp.dot(x_ref[...], w_ref[...], preferred_element_type=jnp.float32)


def _tc_matmul(x, w):
    return pl.pallas_call(
        _mm_body,
        grid=(NPAD // _BLK,),
        in_specs=[
            pl.BlockSpec((_BLK, D), lambda i: (i, 0)),
            pl.BlockSpec((D, H), lambda i: (0, 0)),
        ],
        out_specs=pl.BlockSpec((_BLK, H), lambda i: (i, 0)),
        out_shape=jax.ShapeDtypeStruct((NPAD, H), jnp.float32),
    )(x, w)


def _prep_body(deg2_ref, h1_ref, dinv_ref, h1p_ref):
    deg = deg2_ref[0][:, :16] + deg2_ref[1][:, :16] + 1.0
    dinv = lax.rsqrt(deg)
    dinv_ref[...] = dinv
    h1p_ref[...] = dinv[:, :1] * h1_ref[...]


def _tc_prep(deg2, h1):
    return pl.pallas_call(
        _prep_body,
        out_shape=(
            jax.ShapeDtypeStruct((NPAD, 16), jnp.float32),
            jax.ShapeDtypeStruct((NPAD, H), jnp.float32),
        ),
    )(deg2, h1)


def _mid_body(part_ref, h1_ref, dinv_ref, w2_ref, b1_ref, h2_ref, h2p_ref):
    dinv = dinv_ref[...][:, :1]
    agg = part_ref[0] + part_ref[1]
    z1 = jnp.maximum(dinv * agg + dinv * dinv * h1_ref[...] + b1_ref[...], 0.0)
    h2 = jnp.dot(z1, w2_ref[...], preferred_element_type=jnp.float32)
    h2_ref[...] = h2
    h2p_ref[...] = dinv * h2


def _tc_mid(part1, h1, dinv16, w2, b1):
    return pl.pallas_call(
        _mid_body,
        grid=(NPAD // _BLK,),
        in_specs=[
            pl.BlockSpec((NC, _BLK, H), lambda i: (0, i, 0)),
            pl.BlockSpec((_BLK, H), lambda i: (i, 0)),
            pl.BlockSpec((_BLK, 16), lambda i: (i, 0)),
            pl.BlockSpec((H, H), lambda i: (0, 0)),
            pl.BlockSpec((1, H), lambda i: (0, 0)),
        ],
        out_specs=(
            pl.BlockSpec((_BLK, H), lambda i: (i, 0)),
            pl.BlockSpec((_BLK, H), lambda i: (i, 0)),
        ),
        out_shape=(
            jax.ShapeDtypeStruct((NPAD, H), jnp.float32),
            jax.ShapeDtypeStruct((NPAD, H), jnp.float32),
        ),
    )(part1, h1, dinv16, w2, b1)


def _final_body(part_ref, h2_ref, dinv_ref, b2_ref, batch_ref, wfc_ref, bfc_ref,
                out_ref, acc_ref, cnt_ref):
    i = pl.program_id(0)

    @pl.when(i == 0)
    def _():
        acc_ref[...] = jnp.zeros_like(acc_ref)
        cnt_ref[...] = jnp.zeros_like(cnt_ref)

    dinv = dinv_ref[...][:, :1]
    z2 = dinv * (part_ref[0] + part_ref[1]) + dinv * dinv * h2_ref[...] + b2_ref[...]
    onehot = (batch_ref[...] == lax.broadcasted_iota(jnp.int32, (1, G), 1)
              ).astype(jnp.float32)
    acc_ref[...] += lax.dot_general(onehot, z2, (((0,), (0,)), ((), ())),
                                    preferred_element_type=jnp.float32)
    cnt_ref[...] += lax.dot_general(onehot, jnp.ones_like(z2),
                                    (((0,), (0,)), ((), ())),
                                    preferred_element_type=jnp.float32)

    @pl.when(i == pl.num_programs(0) - 1)
    def _():
        pooled = acc_ref[...] / jnp.maximum(cnt_ref[...], 1.0)
        out_ref[...] = jnp.dot(pooled, wfc_ref[...],
                               preferred_element_type=jnp.float32) + bfc_ref[...]


def _tc_final(part2, h2, dinv16, b2, batch2d, wfc, bfc):
    return pl.pallas_call(
        _final_body,
        grid=(NPAD // _BLK,),
        in_specs=[
            pl.BlockSpec((NC, _BLK, H), lambda i: (0, i, 0)),
            pl.BlockSpec((_BLK, H), lambda i: (i, 0)),
            pl.BlockSpec((_BLK, 16), lambda i: (i, 0)),
            pl.BlockSpec((1, H), lambda i: (0, 0)),
            pl.BlockSpec((_BLK, 1), lambda i: (i, 0)),
            pl.BlockSpec((H, O), lambda i: (0, 0)),
            pl.BlockSpec((1, O), lambda i: (0, 0)),
        ],
        out_specs=pl.BlockSpec((G, O), lambda i: (0, 0)),
        out_shape=jax.ShapeDtypeStruct((G, O), jnp.float32),
        scratch_shapes=[
            pltpu.VMEM((G, H), jnp.float32),
            pltpu.VMEM((G, H), jnp.float32),
        ],
    )(part2, h2, dinv16, b2, batch2d, wfc, bfc)


def kernel(x, edge_index, batch, W1, b1, W2, b2, Wfc, bfc):
    src = edge_index[0]
    dst = edge_index[1]
    pad = EPAD - E
    src_f = jnp.concatenate([src, jnp.zeros((pad,), jnp.int32)])
    dst_f = jnp.concatenate(
        [dst, N + (jnp.arange(pad, dtype=jnp.int32) % (NPAD - N))])
    dst3 = dst_f.reshape(NW, NCH, CH)

    def agg_layout(flat):
        c0 = flat[: NS * NCH0 * CH].reshape(NS, NCH0, CH)
        c1 = flat[NS * NCH0 * CH :].reshape(NS, NCH1, CH)
        c1 = jnp.pad(c1, ((0, 0), (0, NCH0 - NCH1), (0, 0)))
        return jnp.concatenate([c0, c1], axis=0)

    src3a = agg_layout(src_f)
    dst3a = agg_layout(dst_f)

    x_pad = jnp.pad(x, ((0, NPAD - N), (0, 0)))
    batch2d = jnp.pad(batch, (0, NPAD - N), constant_values=G).reshape(NPAD, 1)
    ones128 = jnp.ones((CH, D), jnp.float32)
    zeros128 = jnp.zeros((NPAD, D), jnp.float32)
    b1r = b1.reshape(1, H)
    b2r = b2.reshape(1, H)
    bfcr = bfc.reshape(1, O)

    deg2 = _sc_degree(dst3, ones128, zeros128)
    h1 = _tc_matmul(x_pad, W1)
    dinv16, h1p = _tc_prep(deg2, h1)
    part1 = _sc_aggregate(src3a, dst3a, h1p, zeros128)
    h2, h2p = _tc_mid(part1, h1, dinv16, W2, b1r)
    part2 = _sc_aggregate(src3a, dst3a, h2p, zeros128)
    return _tc_final(part2, h2, dinv16, b2r, batch2d, Wfc, bfcr)


# speedup vs baseline: 1.2294x; 1.0311x over previous
import functools

import jax
import jax.numpy as jnp
from jax import lax
from jax.experimental import pallas as pl
from jax.experimental.pallas import tpu as pltpu
from jax.experimental.pallas import tpu_sc as plsc

N = 10000
E = 320000
D = 128
H = 128
O = 64
G = 16

NC = 2
NS = 16
NW = NC * NS
CH = 128
NCH = 79
NCH0 = 113
NCH1 = 45
EPAD = NW * NCH * CH
NPAD = N + 112
RPS = NPAD // NS
HN = NCH // 2

_MESH = plsc.VectorSubcoreMesh(core_axis_name="c", subcore_axis_name="s")


def _deg_body(dst_hbm, ones_hbm, zeros_hbm, out_hbm, dst_v, ones_v, acc_sh):
    c = lax.axis_index("c")
    s = lax.axis_index("s")
    w = c * NS + s
    pltpu.sync_copy(zeros_hbm.at[pl.ds(s * RPS, RPS)], acc_sh.at[pl.ds(s * RPS, RPS)])
    pltpu.sync_copy(dst_hbm.at[w], dst_v)
    pltpu.sync_copy(ones_hbm, ones_v)
    plsc.subcore_barrier()

    @pl.loop(0, NCH)
    def _(j):
        pltpu.sync_copy(ones_v, acc_sh.at[dst_v.at[j]], add=True)

    plsc.subcore_barrier()
    pltpu.sync_copy(acc_sh.at[pl.ds(s * RPS, RPS)], out_hbm.at[c, pl.ds(s * RPS, RPS)])


def _sc_degree(dst3, ones128, zeros128):
    kern = pl.kernel(
        _deg_body,
        out_type=jax.ShapeDtypeStruct((NC, NPAD, D), jnp.float32),
        mesh=_MESH,
        scratch_types=[
            pltpu.VMEM((NCH, CH), jnp.int32),
            pltpu.VMEM((CH, D), jnp.float32),
            pltpu.VMEM_SHARED((NPAD, D), jnp.float32),
        ],
    )
    return kern(dst3, ones128, zeros128)


def _agg_body(src_hbm, dst_hbm, h_hbm, zeros_hbm, out_hbm,
              src_v, dst_v, rows_v, acc_sh):
    c = lax.axis_index("c")
    s = lax.axis_index("s")
    w = c * NS + s
    pltpu.sync_copy(zeros_hbm.at[pl.ds(s * RPS, RPS)], acc_sh.at[pl.ds(s * RPS, RPS)])
    pltpu.sync_copy(src_hbm.at[w], src_v)
    pltpu.sync_copy(dst_hbm.at[w], dst_v)
    plsc.subcore_barrier()

    @pl.when(c == 0)
    def _():
        @pl.loop(0, NCH0)
        def _(j):
            pltpu.sync_copy(h_hbm.at[src_v.at[j]], rows_v)
            pltpu.sync_copy(rows_v, acc_sh.at[dst_v.at[j]], add=True)

    @pl.when(c == 1)
    def _():
        @pl.loop(0, NCH1)
        def _(j):
            pltpu.sync_copy(h_hbm.at[src_v.at[j]], rows_v)
            pltpu.sync_copy(rows_v, acc_sh.at[dst_v.at[j]], add=True)

    plsc.subcore_barrier()
    pltpu.sync_copy(acc_sh.at[pl.ds(s * RPS, RPS)], out_hbm.at[c, pl.ds(s * RPS, RPS)])


def _sc_aggregate(src3, dst3, h, zeros128):
    kern = pl.kernel(
        _agg_body,
        out_type=jax.ShapeDtypeStruct((NC, NPAD, D), jnp.float32),
        mesh=_MESH,
        scratch_types=[
            pltpu.VMEM((NCH0, CH), jnp.int32),
            pltpu.VMEM((NCH0, CH), jnp.int32),
            pltpu.VMEM((CH, D), jnp.float32),
            pltpu.VMEM_SHARED((NPAD, D), jnp.float32),
        ],
    )
    return kern(src3, dst3, h, zeros128)


_BLK = 2528


def _mm_body(x_ref, w_ref, o_ref):
    o_ref[...] = jnp.dot(x_ref[...], w_ref[...], preferred_element_type=jnp.float32)


def _tc_matmul(x, w):
    return pl.pallas_call(
        _mm_body,
        grid=(NPAD // _BLK,),
        in_specs=[
            pl.BlockSpec((_BLK, D), lambda i: (i, 0)),
            pl.BlockSpec((D, H), lambda i: (0, 0)),
        ],
        out_specs=pl.BlockSpec((_BLK, H), lambda i: (i, 0)),
        out_shape=jax.ShapeDtypeStruct((NPAD, H), jnp.float32),
    )(x, w)


def _prep_body(deg2_ref, h1_ref, dinv_ref, h1p_ref):
    deg = deg2_ref[0][:, :16] + deg2_ref[1][:, :16] + 1.0
    dinv = lax.rsqrt(deg)
    dinv_ref[...] = dinv
    h1p_ref[...] = dinv[:, :1] * h1_ref[...]


def _tc_prep(deg2, h1):
    return pl.pallas_call(
        _prep_body,
        out_shape=(
            jax.ShapeDtypeStruct((NPAD, 16), jnp.float32),
            jax.ShapeDtypeStruct((NPAD, H), jnp.float32),
        ),
    )(deg2, h1)


def _mid_body(part_ref, h1_ref, dinv_ref, w2_ref, b1_ref, h2_ref, h2p_ref):
    dinv = dinv_ref[...][:, :1]
    agg = part_ref[0] + part_ref[1]
    z1 = jnp.maximum(dinv * agg + dinv * dinv * h1_ref[...] + b1_ref[...], 0.0)
    h2 = jnp.dot(z1, w2_ref[...], preferred_element_type=jnp.float32)
    h2_ref[...] = h2
    h2p_ref[...] = dinv * h2


def _tc_mid(part1, h1, dinv16, w2, b1):
    return pl.pallas_call(
        _mid_body,
        grid=(NPAD // _BLK,),
        in_specs=[
            pl.BlockSpec((NC, _BLK, H), lambda i: (0, i, 0)),
            pl.BlockSpec((_BLK, H), lambda i: (i, 0)),
            pl.BlockSpec((_BLK, 16), lambda i: (i, 0)),
            pl.BlockSpec((H, H), lambda i: (0, 0)),
            pl.BlockSpec((1, H), lambda i: (0, 0)),
        ],
        out_specs=(
            pl.BlockSpec((_BLK, H), lambda i: (i, 0)),
            pl.BlockSpec((_BLK, H), lambda i: (i, 0)),
        ),
        out_shape=(
            jax.ShapeDtypeStruct((NPAD, H), jnp.float32),
            jax.ShapeDtypeStruct((NPAD, H), jnp.float32),
        ),
    )(part1, h1, dinv16, w2, b1)


def _final_body(part_ref, h2_ref, dinv_ref, b2_ref, batch_ref, wfc_ref, bfc_ref,
                out_ref, acc_ref, cnt_ref):
    i = pl.program_id(0)

    @pl.when(i == 0)
    def _():
        acc_ref[...] = jnp.zeros_like(acc_ref)
        cnt_ref[...] = jnp.zeros_like(cnt_ref)

    dinv = dinv_ref[...][:, :1]
    z2 = dinv * (part_ref[0] + part_ref[1]) + dinv * dinv * h2_ref[...] + b2_ref[...]
    onehot = (batch_ref[...] == lax.broadcasted_iota(jnp.int32, (1, G), 1)
              ).astype(jnp.float32)
    acc_ref[...] += lax.dot_general(onehot, z2, (((0,), (0,)), ((), ())),
                                    preferred_element_type=jnp.float32)
    cnt_ref[...] += lax.dot_general(onehot, jnp.ones_like(z2),
                                    (((0,), (0,)), ((), ())),
                                    preferred_element_type=jnp.float32)

    @pl.when(i == pl.num_programs(0) - 1)
    def _():
        pooled = acc_ref[...] / jnp.maximum(cnt_ref[...], 1.0)
        out_ref[...] = jnp.dot(pooled, wfc_ref[...],
                               preferred_element_type=jnp.float32) + bfc_ref[...]


def _tc_final(part2, h2, dinv16, b2, batch2d, wfc, bfc):
    return pl.pallas_call(
        _final_body,
        grid=(NPAD // _BLK,),
        in_specs=[
            pl.BlockSpec((NC, _BLK, H), lambda i: (0, i, 0)),
            pl.BlockSpec((_BLK, H), lambda i: (i, 0)),
            pl.BlockSpec((_BLK, 16), lambda i: (i, 0)),
            pl.BlockSpec((1, H), lambda i: (0, 0)),
            pl.BlockSpec((_BLK, 1), lambda i: (i, 0)),
            pl.BlockSpec((H, O), lambda i: (0, 0)),
            pl.BlockSpec((1, O), lambda i: (0, 0)),
        ],
        out_specs=pl.BlockSpec((G, O), lambda i: (0, 0)),
        out_shape=jax.ShapeDtypeStruct((G, O), jnp.float32),
        scratch_shapes=[
            pltpu.VMEM((G, H), jnp.float32),
            pltpu.VMEM((G, H), jnp.float32),
        ],
    )(part2, h2, dinv16, b2, batch2d, wfc, bfc)


def kernel(x, edge_index, batch, W1, b1, W2, b2, Wfc, bfc):
    src = edge_index[0]
    dst = edge_index[1]
    pad = EPAD - E
    src_f = jnp.concatenate([src, jnp.zeros((pad,), jnp.int32)])
    dst_f = jnp.concatenate(
        [dst, N + (jnp.arange(pad, dtype=jnp.int32) % (NPAD - N))])
    dst3 = dst_f.reshape(NW, NCH, CH)

    def agg_layout(flat):
        c0 = flat[: NS * NCH0 * CH].reshape(NS, NCH0, CH)
        c1 = flat[NS * NCH0 * CH :].reshape(NS, NCH1, CH)
        c1 = jnp.pad(c1, ((0, 0), (0, NCH0 - NCH1), (0, 0)))
        return jnp.concatenate([c0, c1], axis=0)

    src3a = agg_layout(src_f)
    dst3a = agg_layout(dst_f)

    x_pad = jnp.pad(x, ((0, NPAD - N), (0, 0)))
    batch2d = jnp.pad(batch, (0, NPAD - N), constant_values=G).reshape(NPAD, 1)
    ones128 = jnp.ones((CH, D), jnp.float32)
    zeros128 = jnp.zeros((NPAD, D), jnp.float32)
    b1r = b1.reshape(1, H)
    b2r = b2.reshape(1, H)
    bfcr = bfc.reshape(1, O)

    deg2 = _sc_degree(dst3, ones128, zeros128)
    h1 = _tc_matmul(x_pad, W1)
    dinv16, h1p = _tc_prep(deg2, h1)
    part1 = _sc_aggregate(src3a, dst3a, h1p, zeros128)
    h2, h2p = _tc_mid(part1, h1, dinv16, W2, b1r)
    part2 = _sc_aggregate(src3a, dst3a, h2p, zeros128)
    return _tc_final(part2, h2, dinv16, b2r, batch2d, Wfc, bfcr)
